# Initial kernel scaffold; baseline (speedup 1.0000x reference)
import functools

import jax
import jax.numpy as jnp
from jax import lax
from jax.experimental import pallas as pl
from jax.experimental.pallas import tpu as pltpu
from jax.experimental.pallas import tpu_sc as plsc

N = 10000
D = 256
K = 3
E = 160000

HALF = 128
NC = 2
NS = 16
L = 16
EPT = E // NS
C = 80
NCH = EPT // C
RPT = N // NS
CW = 16

_mesh = plsc.VectorSubcoreMesh(core_axis_name="c", subcore_axis_name="s")


@functools.partial(
    pl.kernel,
    out_type=(
        jax.ShapeDtypeStruct((NC, K, N, HALF), jnp.float32),
        jax.ShapeDtypeStruct((K, N, CW), jnp.float32),
    ),
    mesh=_mesh,
    scratch_types=[
        pltpu.VMEM((RPT, HALF), jnp.float32),
        pltpu.VMEM((RPT, CW), jnp.float32),
        pltpu.VMEM((C,), jnp.int32),
        pltpu.VMEM((C,), jnp.int32),
        pltpu.VMEM((C,), jnp.int32),
        pltpu.VMEM((C, HALF), jnp.float32),
        pltpu.VMEM((C, CW), jnp.float32),
        pltpu.VMEM_SHARED((N, HALF), jnp.float32),
        pltpu.VMEM_SHARED((N, CW), jnp.float32),
        pltpu.SemaphoreType.DMA,
    ],
)
def _sc_segment(edges_hbm, x2_hbm, z128_hbm, z16_hbm, ones_hbm,
                s_out, cnt_out,
                zbuf, zbuf16, src_v, gidx_v, dst_v, rows_v, ones_v,
                shared, cshared, sem):
    cid = lax.axis_index("c")
    sid = lax.axis_index("s")
    r0 = pl.multiple_of(sid * RPT, 8)

    pltpu.sync_copy(z128_hbm, zbuf)
    pltpu.sync_copy(z16_hbm, zbuf16)
    pltpu.sync_copy(ones_hbm, ones_v)

    for k in range(K):
        kbase = k * N

        pltpu.sync_copy(zbuf, shared.at[pl.ds(r0, RPT)])

        @pl.when(cid == 0)
        def _():
            pltpu.sync_copy(zbuf16, cshared.at[pl.ds(r0, RPT)])

        plsc.subcore_barrier()

        def body(i, carry):
            e0 = pl.multiple_of(sid * EPT + i * C, 8)
            pltpu.sync_copy(edges_hbm.at[k, 0, pl.ds(e0, C)], src_v)
            pltpu.sync_copy(edges_hbm.at[k, 1, pl.ds(e0, C)], dst_v)
            for j in range(C // L):
                s16 = src_v[pl.ds(j * L, L)]
                gidx_v[pl.ds(j * L, L)] = (s16 + kbase) * 2 + cid
            pltpu.async_copy(x2_hbm.at[gidx_v], rows_v, sem).wait()
            pltpu.sync_copy(rows_v, shared.at[dst_v], add=True)

            @pl.when(cid == 0)
            def _():
                pltpu.sync_copy(ones_v, cshared.at[dst_v], add=True)

            return carry

        lax.fori_loop(0, NCH, body, 0)
        plsc.subcore_barrier()

        pltpu.sync_copy(shared.at[pl.ds(r0, RPT)],
                        s_out.at[cid, k, pl.ds(r0, RPT)])

        @pl.when(cid == 0)
        def _():
            pltpu.sync_copy(cshared.at[pl.ds(r0, RPT)],
                            cnt_out.at[k, pl.ds(r0, RPT)])

        plsc.subcore_barrier()


def _prep_body(wl_ref, bl_ref, wr_ref, wk_ref, a_ref, b_ref, c_ref):
    for k in range(K):
        top = wk_ref[k, :D, :]
        bot = wk_ref[k, D:, :]
        a_ref[k] = top + jnp.dot(wr_ref[...], bot,
                                 preferred_element_type=jnp.float32)
        b_ref[k] = jnp.dot(wl_ref[...], bot,
                           preferred_element_type=jnp.float32)
    bsum = wk_ref[0, D:, :] + wk_ref[1, D:, :] + wk_ref[2, D:, :]
    c_ref[...] = jnp.dot(bl_ref[...], bsum,
                         preferred_element_type=jnp.float32)


_prep_weights = pl.pallas_call(
    _prep_body,
    out_shape=(
        jax.ShapeDtypeStruct((K, D, D), jnp.float32),
        jax.ShapeDtypeStruct((K, D, D), jnp.float32),
        jax.ShapeDtypeStruct((1, D), jnp.float32),
    ),
)

R_BLK = 1000


def _dense_body(x_ref, sl_ref, sr_ref, cnt_ref, a_ref, b_ref, c_ref, o_ref):
    acc = jnp.broadcast_to(c_ref[...], (R_BLK, D))
    for k in range(K):
        m = jnp.maximum(cnt_ref[k][:, 0:1], 1.0)
        agg_l = sl_ref[k] / m
        agg_r = sr_ref[k] / m
        acc = acc + jnp.dot(x_ref[k], a_ref[k],
                            preferred_element_type=jnp.float32)
        acc = acc + jnp.dot(agg_l, b_ref[k, :HALF, :],
                            preferred_element_type=jnp.float32)
        acc = acc + jnp.dot(agg_r, b_ref[k, HALF:, :],
                            preferred_element_type=jnp.float32)
    h = jnp.maximum(acc, 0.0)
    norm = jnp.sqrt(jnp.sum(h * h, axis=1, keepdims=True))
    o_ref[...] = h / jnp.maximum(norm, 1e-12)


_dense = pl.pallas_call(
    _dense_body,
    grid=(N // R_BLK,),
    in_specs=[
        pl.BlockSpec((K, R_BLK, D), lambda i: (0, i, 0)),
        pl.BlockSpec((K, R_BLK, HALF), lambda i: (0, i, 0)),
        pl.BlockSpec((K, R_BLK, HALF), lambda i: (0, i, 0)),
        pl.BlockSpec((K, R_BLK, CW), lambda i: (0, i, 0)),
        pl.BlockSpec((K, D, D), lambda i: (0, 0, 0)),
        pl.BlockSpec((K, D, D), lambda i: (0, 0, 0)),
        pl.BlockSpec((1, D), lambda i: (0, 0)),
    ],
    out_specs=pl.BlockSpec((R_BLK, D), lambda i: (i, 0)),
    out_shape=jax.ShapeDtypeStruct((N, D), jnp.float32),
)


@jax.jit
def kernel(H_K_prev, edgelists, W_l, b_l, W_r, W_ks):
    x2 = H_K_prev.reshape(K * N * 2, HALF)
    z128 = jnp.zeros((RPT, HALF), jnp.float32)
    z16 = jnp.zeros((RPT, CW), jnp.float32)
    ones = jnp.ones((C, CW), jnp.float32)
    s_out, cnt_out = _sc_segment(edgelists, x2, z128, z16, ones)
    a_w, b_w, c_w = _prep_weights(W_l, b_l.reshape(1, D), W_r, W_ks)
    out = _dense(H_K_prev, s_out[0], s_out[1], cnt_out, a_w, b_w, c_w)
    return out[None]


# speedup vs baseline: 2.9232x; 2.9232x over previous
import functools

import jax
import jax.numpy as jnp
from jax import lax
from jax.experimental import pallas as pl
from jax.experimental.pallas import tpu as pltpu
from jax.experimental.pallas import tpu_sc as plsc

N = 10000
D = 256
K = 3
E = 160000

HALF = 128
NC = 2
NS = 16
L = 16
EPT = E // NS
C = 80
NCH = EPT // C
NPAD = 10240
RPT = NPAD // NS
CW = 16


@functools.cache
def _get_sc_segment():
  mesh = plsc.VectorSubcoreMesh(core_axis_name="c", subcore_axis_name="s",
                                num_cores=NC, num_subcores=NS)

  @functools.partial(
      pl.kernel,
      out_type=(
          jax.ShapeDtypeStruct((NC, K, NPAD, HALF), jnp.float32),
          jax.ShapeDtypeStruct((K, NPAD, CW), jnp.float32),
      ),
      mesh=mesh,
      compiler_params=pltpu.CompilerParams(use_tc_tiling_on_sc=False),
      scratch_types=[
          pltpu.VMEM((C,), jnp.int32),
          pltpu.VMEM((C,), jnp.int32),
          pltpu.VMEM((C,), jnp.int32),
          pltpu.VMEM((C, HALF), jnp.float32),
          pltpu.VMEM((C, CW), jnp.float32),
          pltpu.VMEM_SHARED((NPAD, HALF), jnp.float32),
          pltpu.VMEM_SHARED((NPAD, CW), jnp.float32),
          pltpu.SemaphoreType.DMA,
      ],
  )
  def _sc_segment(edges_hbm, x2_hbm, z128_hbm, z16_hbm, ones_hbm,
                  s_out, cnt_out,
                  src_v, gidx_v, dst_v, rows_v, ones_v,
                  shared, cshared, sem):
    cid = lax.axis_index("c")
    sid = lax.axis_index("s")
    r0 = pl.multiple_of(sid * RPT, 8)

    pltpu.sync_copy(ones_hbm, ones_v)

    for k in range(K):
      kbase = k * N
      src_base = (2 * k) * E
      dst_base = (2 * k + 1) * E

      pltpu.sync_copy(z128_hbm.at[pl.ds(r0, RPT)], shared.at[pl.ds(r0, RPT)])

      @pl.when(cid == 0)
      def _():
        pltpu.sync_copy(z16_hbm.at[pl.ds(r0, RPT)],
                        cshared.at[pl.ds(r0, RPT)])

      plsc.subcore_barrier()

      def body(i, carry):
        e0 = pl.multiple_of(sid * EPT + i * C, 8)
        pltpu.sync_copy(edges_hbm.at[pl.ds(src_base + e0, C)], src_v)
        pltpu.sync_copy(edges_hbm.at[pl.ds(dst_base + e0, C)], dst_v)
        for j in range(C // L):
          s16 = src_v[pl.ds(j * L, L)]
          gidx_v[pl.ds(j * L, L)] = (s16 + kbase) * 2 + cid
        pltpu.async_copy(x2_hbm.at[gidx_v], rows_v, sem).wait()
        pltpu.sync_copy(rows_v, shared.at[dst_v], add=True)

        @pl.when(cid == 0)
        def _():
          pltpu.sync_copy(ones_v, cshared.at[dst_v], add=True)

        return carry

      lax.fori_loop(0, NCH, body, 0)
      plsc.subcore_barrier()

      pltpu.sync_copy(shared.at[pl.ds(r0, RPT)],
                      s_out.at[cid, k, pl.ds(r0, RPT)])

      @pl.when(cid == 0)
      def _():
        pltpu.sync_copy(cshared.at[pl.ds(r0, RPT)],
                        cnt_out.at[k, pl.ds(r0, RPT)])

      plsc.subcore_barrier()

  return _sc_segment


def _prep_body(wl_ref, bl_ref, wr_ref, wk_ref, a_ref, b_ref, c_ref):
    for k in range(K):
        top = wk_ref[k, :D, :]
        bot = wk_ref[k, D:, :]
        a_ref[k] = top + jnp.dot(wr_ref[...], bot,
                                 preferred_element_type=jnp.float32)
        b_ref[k] = jnp.dot(wl_ref[...], bot,
                           preferred_element_type=jnp.float32)
    bsum = wk_ref[0, D:, :] + wk_ref[1, D:, :] + wk_ref[2, D:, :]
    c_ref[...] = jnp.dot(bl_ref[...], bsum,
                         preferred_element_type=jnp.float32)


_prep_weights = pl.pallas_call(
    _prep_body,
    out_shape=(
        jax.ShapeDtypeStruct((K, D, D), jnp.float32),
        jax.ShapeDtypeStruct((K, D, D), jnp.float32),
        jax.ShapeDtypeStruct((1, D), jnp.float32),
    ),
)

R_BLK = 2000


def _dense_body(x_ref, sl_ref, sr_ref, cnt_ref, a_ref, b_ref, c_ref, o_ref):
    acc = jnp.broadcast_to(c_ref[...], (R_BLK, D))
    for k in range(K):
        m = jnp.maximum(cnt_ref[k][:, 0:1], 1.0)
        agg_l = sl_ref[k] / m
        agg_r = sr_ref[k] / m
        acc = acc + jnp.dot(x_ref[k], a_ref[k],
                            preferred_element_type=jnp.float32)
        acc = acc + jnp.dot(agg_l, b_ref[k, :HALF, :],
                            preferred_element_type=jnp.float32)
        acc = acc + jnp.dot(agg_r, b_ref[k, HALF:, :],
                            preferred_element_type=jnp.float32)
    h = jnp.maximum(acc, 0.0)
    norm = jnp.sqrt(jnp.sum(h * h, axis=1, keepdims=True))
    o_ref[...] = h / jnp.maximum(norm, 1e-12)


_dense = pl.pallas_call(
    _dense_body,
    grid=(N // R_BLK,),
    in_specs=[
        pl.BlockSpec((K, R_BLK, D), lambda i: (0, i, 0)),
        pl.BlockSpec((K, R_BLK, HALF), lambda i: (0, i, 0)),
        pl.BlockSpec((K, R_BLK, HALF), lambda i: (0, i, 0)),
        pl.BlockSpec((K, R_BLK, CW), lambda i: (0, i, 0)),
        pl.BlockSpec((K, D, D), lambda i: (0, 0, 0)),
        pl.BlockSpec((K, D, D), lambda i: (0, 0, 0)),
        pl.BlockSpec((1, D), lambda i: (0, 0)),
    ],
    out_specs=pl.BlockSpec((R_BLK, D), lambda i: (i, 0)),
    out_shape=jax.ShapeDtypeStruct((N, D), jnp.float32),
)


@jax.jit
def kernel(H_K_prev, edgelists, W_l, b_l, W_r, W_ks):
    x2 = H_K_prev.reshape(K * N * 2, HALF)
    edges_flat = edgelists.reshape(K * 2 * E)
    z128 = jnp.zeros((NPAD, HALF), jnp.float32)
    z16 = jnp.zeros((NPAD, CW), jnp.float32)
    ones = jnp.ones((C, CW), jnp.float32)
    s_out, cnt_out = _get_sc_segment()(edges_flat, x2, z128, z16, ones)
    a_w, b_w, c_w = _prep_weights(W_l, b_l.reshape(1, D), W_r, W_ks)
    out = _dense(H_K_prev, s_out[0], s_out[1], cnt_out, a_w, b_w, c_w)
    return out[None]
